# baseline (speedup 1.0000x reference)
import functools

import jax
import jax.numpy as jnp
from jax import lax
from jax.experimental import pallas as pl
from jax.experimental.pallas import tpu as pltpu
from jax.experimental.pallas import tpu_sc as plsc

N = 1048576
NC = 2
NS = 16
NW = NC * NS
C = N // NW
LANES = 16
UNROLL = 8


def _l1_body(preds_hbm, targets_hbm, out_hbm, p_v, t_v, acc_v, psem, tsem):
    c = lax.axis_index("c")
    s = lax.axis_index("s")
    wid = s * NC + c
    base = wid * C
    cp_p = pltpu.async_copy(preds_hbm.at[pl.ds(base, C)], p_v, psem)
    cp_t = pltpu.async_copy(targets_hbm.at[pl.ds(2 * base, 2 * C)], t_v, tsem)
    cp_p.wait()
    cp_t.wait()

    eidx = lax.broadcasted_iota(jnp.int32, (LANES,), 0) * 2
    zero = jnp.zeros((LANES,), jnp.float32)

    def body(i, accs):
        off = i * (LANES * UNROLL)
        new = []
        for j in range(UNROLL):
            o = off + j * LANES
            p = p_v[pl.ds(o, LANES)]
            t0 = plsc.load_gather(t_v, [eidx + 2 * o])
            new.append(accs[j] + jnp.abs(p - t0))
        return tuple(new)

    accs = lax.fori_loop(0, C // (LANES * UNROLL), body, (zero,) * UNROLL)
    total = accs[0]
    for j in range(1, UNROLL):
        total = total + accs[j]
    acc_v[...] = total * (1.0 / N)
    pltpu.sync_copy(acc_v, out_hbm.at[wid])


_mesh = plsc.VectorSubcoreMesh(core_axis_name="c", subcore_axis_name="s")

_l1_partials = functools.partial(
    pl.kernel,
    mesh=_mesh,
    compiler_params=pltpu.CompilerParams(needs_layout_passes=False),
    out_type=jax.ShapeDtypeStruct((NW, LANES), jnp.float32),
    scratch_types=[
        pltpu.VMEM((C,), jnp.float32),
        pltpu.VMEM((2 * C,), jnp.float32),
        pltpu.VMEM((LANES,), jnp.float32),
        pltpu.SemaphoreType.DMA,
        pltpu.SemaphoreType.DMA,
    ],
)(_l1_body)


@jax.jit
def kernel(preds1, targets1):
    partials = _l1_partials(preds1, jnp.reshape(targets1, (2 * N,)))
    loss = jnp.sum(partials)
    return loss, jnp.reshape(loss, (1,))


# speedup vs baseline: 45.8560x; 45.8560x over previous
import functools

import jax
import jax.numpy as jnp
from jax import lax
from jax.experimental import pallas as pl
from jax.experimental.pallas import tpu as pltpu
from jax.experimental.pallas import tpu_sc as plsc

N = 1048576
NC = 2
NS = 16
NW = NC * NS
C = N // NW
LANES = 16
UNROLL = 8
BLK = 128


def _l1_body(preds_hbm, targets_hbm, out_hbm, p_v, t_v, acc_v, psem, tsem):
    c = lax.axis_index("c")
    s = lax.axis_index("s")
    wid = s * NC + c
    base = wid * C
    cp_p = pltpu.async_copy(preds_hbm.at[pl.ds(base, C)], p_v, psem)
    cp_t = pltpu.async_copy(targets_hbm.at[pl.ds(2 * base, 2 * C)], t_v, tsem)
    cp_p.wait()
    cp_t.wait()

    zero = jnp.zeros((LANES,), jnp.float32)

    def body(i, accs):
        new = []
        for j in range(UNROLL):
            p = p_v[pl.ds(i * BLK + j * LANES, LANES)]
            t0 = t_v[pl.ds(i * 2 * BLK + j * LANES, LANES)]
            new.append(accs[j] + jnp.abs(p - t0))
        return tuple(new)

    accs = lax.fori_loop(0, C // BLK, body, (zero,) * UNROLL)
    total = accs[0]
    for j in range(1, UNROLL):
        total = total + accs[j]
    acc_v[...] = total * (1.0 / N)
    pltpu.sync_copy(acc_v, out_hbm.at[wid])


_mesh = plsc.VectorSubcoreMesh(core_axis_name="c", subcore_axis_name="s")

_l1_partials = functools.partial(
    pl.kernel,
    mesh=_mesh,
    compiler_params=pltpu.CompilerParams(needs_layout_passes=False),
    out_type=jax.ShapeDtypeStruct((NW, LANES), jnp.float32),
    scratch_types=[
        pltpu.VMEM((C,), jnp.float32),
        pltpu.VMEM((2 * C,), jnp.float32),
        pltpu.VMEM((LANES,), jnp.float32),
        pltpu.SemaphoreType.DMA,
        pltpu.SemaphoreType.DMA,
    ],
)(_l1_body)


@jax.jit
def kernel(preds1, targets1):
    tflat = jnp.reshape(
        jnp.transpose(jnp.reshape(targets1, (N // BLK, BLK, 2)), (0, 2, 1)),
        (2 * N,),
    )
    partials = _l1_partials(preds1, tflat)
    loss = jnp.sum(partials)
    return loss, jnp.reshape(loss, (1,))


# speedup vs baseline: 46.0983x; 1.0053x over previous
import functools

import jax
import jax.numpy as jnp
from jax import lax
from jax.experimental import pallas as pl
from jax.experimental.pallas import tpu as pltpu
from jax.experimental.pallas import tpu_sc as plsc

N = 1048576
NC = 2
NS = 16
NW = NC * NS
C = N // NW
LANES = 16
UNROLL = 8
BLK = 128


def _l1_body(preds_hbm, targets_hbm, out_hbm, p_v, t_v, acc_v, psem, tsem):
    c = lax.axis_index("c")
    s = lax.axis_index("s")
    wid = s * NC + c
    base = wid * C
    cp_p = pltpu.async_copy(preds_hbm.at[pl.ds(base, C)], p_v, psem)
    cp_t = pltpu.async_copy(targets_hbm.at[pl.ds(2 * base, 2 * C)], t_v, tsem)
    cp_p.wait()
    cp_t.wait()

    zero = jnp.zeros((LANES,), jnp.float32)

    def body(i, accs):
        new = []
        for j in range(UNROLL):
            p = p_v[pl.ds(i * BLK + j * LANES, LANES)]
            t0 = t_v[pl.ds(i * 2 * BLK + j * LANES, LANES)]
            new.append(accs[j] + jnp.abs(p - t0))
        return tuple(new)

    accs = lax.fori_loop(0, C // BLK, body, (zero,) * UNROLL)
    total = accs[0]
    for j in range(1, UNROLL):
        total = total + accs[j]
    acc_v[...] = total * (1.0 / N)
    pltpu.sync_copy(acc_v, out_hbm.at[wid])


_mesh = plsc.VectorSubcoreMesh(core_axis_name="c", subcore_axis_name="s")

_l1_partials = functools.partial(
    pl.kernel,
    mesh=_mesh,
    compiler_params=pltpu.CompilerParams(
        needs_layout_passes=False,
        skip_device_barrier=True,
        disable_bounds_checks=True,
        disable_semaphore_checks=True,
    ),
    out_type=jax.ShapeDtypeStruct((NW, LANES), jnp.float32),
    scratch_types=[
        pltpu.VMEM((C,), jnp.float32),
        pltpu.VMEM((2 * C,), jnp.float32),
        pltpu.VMEM((LANES,), jnp.float32),
        pltpu.SemaphoreType.DMA,
        pltpu.SemaphoreType.DMA,
    ],
)(_l1_body)


@jax.jit
def kernel(preds1, targets1):
    tflat = jnp.reshape(
        jnp.transpose(jnp.reshape(targets1, (N // BLK, BLK, 2)), (0, 2, 1)),
        (2 * N,),
    )
    partials = _l1_partials(preds1, tflat)
    loss = jnp.sum(partials)
    return loss, jnp.reshape(loss, (1,))


# speedup vs baseline: 48.5523x; 1.0532x over previous
import functools

import jax
import jax.numpy as jnp
from jax import lax
from jax.experimental import pallas as pl
from jax.experimental.pallas import tpu as pltpu
from jax.experimental.pallas import tpu_sc as plsc

N = 1048576
NC = 2
NS = 16
NW = NC * NS
C = N // NW
LANES = 16
UNROLL = 8
BLK = 128
G = C // BLK


def _l1_body(preds_hbm, targets_hbm, out_hbm, p_v, t_v, acc_v, psem, tsem):
    c = lax.axis_index("c")
    s = lax.axis_index("s")
    wid = s * NC + c
    base = wid * C
    cp_p = pltpu.async_copy(preds_hbm.at[pl.ds(base, C)], p_v, psem)
    cp_t = pltpu.async_copy(targets_hbm.at[pl.ds(wid * G, G), 0, :], t_v, tsem)
    cp_p.wait()
    cp_t.wait()

    zero = jnp.zeros((LANES,), jnp.float32)

    def body(i, accs):
        new = []
        for j in range(UNROLL):
            o = i * BLK + j * LANES
            p = p_v[pl.ds(o, LANES)]
            t0 = t_v[i, pl.ds(j * LANES, LANES)]
            new.append(accs[j] + jnp.abs(p - t0))
        return tuple(new)

    accs = lax.fori_loop(0, G, body, (zero,) * UNROLL)
    total = accs[0]
    for j in range(1, UNROLL):
        total = total + accs[j]
    acc_v[...] = total * (1.0 / N)
    pltpu.sync_copy(acc_v, out_hbm.at[wid])


_mesh = plsc.VectorSubcoreMesh(core_axis_name="c", subcore_axis_name="s")

_l1_partials = functools.partial(
    pl.kernel,
    mesh=_mesh,
    compiler_params=pltpu.CompilerParams(
        needs_layout_passes=False,
        skip_device_barrier=True,
        disable_bounds_checks=True,
        disable_semaphore_checks=True,
    ),
    out_type=jax.ShapeDtypeStruct((NW, LANES), jnp.float32),
    scratch_types=[
        pltpu.VMEM((C,), jnp.float32),
        pltpu.VMEM((G, BLK), jnp.float32),
        pltpu.VMEM((LANES,), jnp.float32),
        pltpu.SemaphoreType.DMA,
        pltpu.SemaphoreType.DMA,
    ],
)(_l1_body)


@jax.jit
def kernel(preds1, targets1):
    t3 = jnp.transpose(jnp.reshape(targets1, (N // BLK, BLK, 2)), (0, 2, 1))
    partials = _l1_partials(preds1, t3)
    loss = jnp.sum(partials)
    return loss, jnp.reshape(loss, (1,))
